# baseline (speedup 1.0000x reference)
import functools

import jax
import jax.numpy as jnp
from jax import lax
from jax.experimental import pallas as pl
from jax.experimental.pallas import tpu as pltpu
from jax.experimental.pallas import tpu_sc as plsc

H, W = 2048, 2048
N = H * W
K = N // 4
JMAX = N - 1

NC, NS, LANES = 2, 16, 16
NW = NC * NS
PER_W = N // NW
CH = 16384
NCHUNK = PER_W // CH
NPAIR = NCHUNK // 2

NB = 4096
NB5 = 16

ROWS_BLK = 128
GRID = H // ROWS_BLK

_f32 = jnp.float32
_i32 = jnp.int32


def _t1_body(x_ref, s_ref):
    @pl.when(pl.program_id(0) == 0)
    def _():
        s_ref[0, 0] = 0.0

    s_ref[0, 0] += jnp.sum(jax.nn.sigmoid(5.0 * x_ref[...]))


def _t1(logits):
    return pl.pallas_call(
        _t1_body,
        grid=(GRID,),
        in_specs=[pl.BlockSpec((ROWS_BLK, W), lambda i: (i, 0))],
        out_specs=pl.BlockSpec((1, 1), lambda i: (0, 0),
                               memory_space=pltpu.MemorySpace.SMEM),
        out_shape=jax.ShapeDtypeStruct((1, 1), _f32),
    )(logits)


def _t2_body(l_ref, t_ref, s_ref, renorm_ref, key_ref):
    xbar = s_ref[0, 0] * (1.0 / N)
    sparsity = jnp.float32(K / N)
    r = sparsity / xbar
    beta = (1.0 - sparsity) / (1.0 - xbar)
    prob = jax.nn.sigmoid(5.0 * l_ref[...])
    renorm = jnp.where(r <= 1.0, prob * r, 1.0 - (1.0 - prob) * beta)
    renorm_ref[...] = renorm
    sm = jax.nn.sigmoid(12.0 * (renorm - t_ref[...]))
    key_ref[...] = lax.bitcast_convert_type(sm, _i32)


def _t2(logits, thresh, xsum):
    return pl.pallas_call(
        _t2_body,
        grid=(GRID,),
        in_specs=[
            pl.BlockSpec((ROWS_BLK, W), lambda i: (i, 0)),
            pl.BlockSpec((ROWS_BLK, W), lambda i: (i, 0)),
            pl.BlockSpec((1, 1), lambda i: (0, 0),
                         memory_space=pltpu.MemorySpace.SMEM),
        ],
        out_specs=[
            pl.BlockSpec((ROWS_BLK, W), lambda i: (i, 0)),
            pl.BlockSpec((ROWS_BLK, W), lambda i: (i, 0)),
        ],
        out_shape=[
            jax.ShapeDtypeStruct((H, W), _f32),
            jax.ShapeDtypeStruct((H, W), _i32),
        ],
    )(logits, thresh, xsum)


def _sc_level(level, nb):
    mesh = plsc.VectorSubcoreMesh(core_axis_name="c", subcore_axis_name="s")

    def body(*args):
        if level == 1:
            keys_hbm, hist_hbm, kbuf0, kbuf1, hist, sem0, sem1 = args
            sbuf = None
        else:
            (keys_hbm, state_hbm, hist_hbm, kbuf0, kbuf1, sbuf, hist,
             sem0, sem1) = args
        wid = lax.axis_index("s") * NC + lax.axis_index("c")
        base = wid * PER_W
        lane = lax.iota(_i32, LANES)
        ones = jnp.ones((LANES,), _i32)

        if level > 1:
            pltpu.sync_copy(state_hbm, sbuf)
            sv = sbuf[pl.ds(0, LANES)]
            p1 = sv[0]
            p2 = sv[1]
            p3 = sv[2]
            p4 = sv[3]
            q2 = (p1 << 12) | p2
            v_full = (q2 << 6) | (p3 >> 6)
            j18 = ((p3 & 0x3F) << 12) | p4

        def zero_body(i, _):
            hist[0, pl.ds(i * LANES, LANES)] = jnp.zeros((LANES,), _i32)
            return 0

        lax.fori_loop(0, (LANES * nb) // LANES, zero_body, 0)

        zero16 = jnp.zeros((LANES,), _i32)
        jconst = (JMAX - base) - lane

        UNROLL = 8

        def process(cbuf, c):
            def vec_body(vb, _):
                for u in range(UNROLL):
                    v = vb * UNROLL + u
                    kv = cbuf[pl.ds(v * LANES, LANES)]
                    jdx = jconst - (c * CH + v * LANES)
                    if level == 1:
                        bucket = kv >> 18
                        pred = None
                    elif level == 2:
                        bucket = (kv >> 6) & 0xFFF
                        pred = (kv >> 18) == p1
                    elif level == 3:
                        bucket = ((kv & 0x3F) << 6) | (jdx >> 16)
                        pred = (kv >> 6) == q2
                    elif level == 4:
                        bucket = (jdx >> 4) & 0xFFF
                        pred = (kv == v_full) & ((jdx >> 16) == (p3 & 0x3F))
                    else:
                        bucket = jdx & 0xF
                        pred = (kv == v_full) & ((jdx >> 4) == j18)
                    plsc.addupdate_scatter(hist,
                                           [zero16, (bucket << 4) + lane],
                                           ones, mask=pred)
                return 0

            lax.fori_loop(0, CH // LANES // UNROLL, vec_body, 0)

        pltpu.async_copy(keys_hbm.at[pl.ds(base, CH)], kbuf0, sem0)

        def pair_body(p, _):
            c0 = p * 2
            pltpu.async_copy(
                keys_hbm.at[pl.ds(base + (c0 + 1) * CH, CH)], kbuf1, sem1)
            pltpu.make_async_copy(
                keys_hbm.at[pl.ds(base + c0 * CH, CH)], kbuf0, sem0).wait()
            process(kbuf0, c0)

            @pl.when(p + 1 < NPAIR)
            def _():
                pltpu.async_copy(
                    keys_hbm.at[pl.ds(base + (c0 + 2) * CH, CH)], kbuf0, sem0)

            pltpu.make_async_copy(
                keys_hbm.at[pl.ds(base + (c0 + 1) * CH, CH)], kbuf1,
                sem1).wait()
            process(kbuf1, c0 + 1)
            return 0

        lax.fori_loop(0, NPAIR, pair_body, 0)
        pltpu.sync_copy(hist, hist_hbm.at[pl.ds(wid, 1)])

    scratch = [
        pltpu.VMEM((CH,), _i32),
        pltpu.VMEM((CH,), _i32),
        pltpu.VMEM((1, LANES * nb), _i32),
        pltpu.SemaphoreType.DMA,
        pltpu.SemaphoreType.DMA,
    ]
    if level > 1:
        scratch.insert(2, pltpu.VMEM((128,), _i32))
    return pl.kernel(
        body,
        out_type=jax.ShapeDtypeStruct((NW, LANES * nb), _i32),
        mesh=mesh,
        scratch_types=scratch,
        compiler_params=pltpu.CompilerParams(needs_layout_passes=False,
                                             disable_bounds_checks=True),
    )


def _merge_body(slot, bits, nb, is_l3, hist_ref, sin_ref, sout_ref):
    wrows = (LANES * nb) // 128
    gcols = 128 // LANES
    krem = sin_ref[5]
    acc = jnp.zeros((wrows, 128), _i32)
    for w in range(NW):
        acc = acc + hist_ref[pl.ds(w * wrows, wrows), :]
    bid = (lax.broadcasted_iota(_i32, (wrows, 128), 0) * gcols
           + lax.broadcasted_iota(_i32, (wrows, 128), 1) // LANES)
    cand = jnp.int32(0)
    for bit in reversed(range(bits)):
        t = cand | (1 << bit)
        c = jnp.sum(jnp.where(bid >= t, acc, 0))
        cand = jnp.where(c >= krem, t, cand)
    gt = jnp.sum(jnp.where(bid > cand, acc, 0))
    krem_new = krem - gt
    for j in range(8):
        sout_ref[j] = sin_ref[j]
    sout_ref[slot] = cand
    sout_ref[5] = krem_new
    if is_l3:
        e_cnt = jnp.sum(jnp.where(bid == cand, acc, 0))
        sout_ref[3] = 0
        sout_ref[4] = 0
        sout_ref[7] = (e_cnt > krem_new).astype(_i32)


def _merge(hist, state, slot, bits, nb, is_l3=False):
    wrows = (LANES * nb) // 128
    return pl.pallas_call(
        functools.partial(_merge_body, slot, bits, nb, is_l3),
        in_specs=[
            pl.BlockSpec(memory_space=pltpu.MemorySpace.VMEM),
            pl.BlockSpec(memory_space=pltpu.MemorySpace.SMEM),
        ],
        out_specs=pl.BlockSpec(memory_space=pltpu.MemorySpace.SMEM),
        out_shape=jax.ShapeDtypeStruct((128,), _i32),
    )(hist.reshape(NW * wrows, 128), state)


def _t3_body(k_ref, s_ref, o_ref):
    p1 = s_ref[0]
    p2 = s_ref[1]
    p3 = s_ref[2]
    p4 = s_ref[3]
    p5 = s_ref[4]
    v_full = (((p1 << 12) | p2) << 6) | (p3 >> 6)
    j_thr = ((p3 & 0x3F) << 16) | (p4 << 4) | p5
    i0 = pl.program_id(0)
    r = lax.broadcasted_iota(_i32, (ROWS_BLK, W), 0) + i0 * ROWS_BLK
    c = lax.broadcasted_iota(_i32, (ROWS_BLK, W), 1)
    jdx = JMAX - (r * W + c)
    kv = k_ref[...]
    sel = (kv > v_full) | ((kv == v_full) & (jdx >= j_thr))
    o_ref[...] = sel.astype(_f32)


def _t3(keys2d, state):
    return pl.pallas_call(
        _t3_body,
        grid=(GRID,),
        in_specs=[
            pl.BlockSpec((ROWS_BLK, W), lambda i: (i, 0)),
            pl.BlockSpec((128,), lambda i: (0,),
                         memory_space=pltpu.MemorySpace.SMEM),
        ],
        out_specs=pl.BlockSpec((ROWS_BLK, W), lambda i: (i, 0)),
        out_shape=jax.ShapeDtypeStruct((H, W), _f32),
    )(keys2d, state)


@functools.lru_cache(maxsize=None)
def _sc(level, nb):
    return _sc_level(level, nb)


def kernel(x, prob_mask_logits, thresh):
    del x
    xsum = _t1(prob_mask_logits)
    renorm, keys2d = _t2(prob_mask_logits, thresh, xsum)
    keys_flat = keys2d.reshape(-1)

    state0 = jnp.zeros((128,), _i32).at[5].set(K)
    s1 = _merge(_sc(1, NB)(keys_flat), state0, slot=0, bits=12, nb=NB)
    s2 = _merge(_sc(2, NB)(keys_flat, s1), s1, slot=1, bits=12, nb=NB)
    s3 = _merge(_sc(3, NB)(keys_flat, s2), s2, slot=2, bits=12, nb=NB,
                is_l3=True)

    def tie_path(args):
        kf, s = args
        s4 = _merge(_sc(4, NB)(kf, s), s, slot=3, bits=12, nb=NB)
        s5 = _merge(_sc(5, NB5)(kf, s4), s4, slot=4, bits=4, nb=NB5)
        return s5

    sfin = lax.cond(s3[7] > 0, tie_path, lambda a: a[1], (keys_flat, s3))
    hard = _t3(keys2d, sfin)
    return hard, renorm


# speedup vs baseline: 1.1467x; 1.1467x over previous
import functools

import jax
import jax.numpy as jnp
from jax import lax
from jax.experimental import pallas as pl
from jax.experimental.pallas import tpu as pltpu
from jax.experimental.pallas import tpu_sc as plsc

H, W = 2048, 2048
N = H * W
K = N // 4
JMAX = N - 1

NC, NS, LANES = 2, 16, 16
NW = NC * NS
PER_W = N // NW
ROWS_W = PER_W // W
CH = 16384
CHR = CH // W
VPR = W // LANES
NCHUNK = PER_W // CH
NPAIR = NCHUNK // 2

NB = 4096
NB5 = 16

ROWS_BLK = 128
GRID = H // ROWS_BLK

_f32 = jnp.float32
_i32 = jnp.int32


def _t1_body(x_ref, s_ref):
    @pl.when(pl.program_id(0) == 0)
    def _():
        s_ref[0, 0] = 0.0

    s_ref[0, 0] += jnp.sum(jax.nn.sigmoid(5.0 * x_ref[...]))


def _t1(logits):
    return pl.pallas_call(
        _t1_body,
        grid=(GRID,),
        in_specs=[pl.BlockSpec((ROWS_BLK, W), lambda i: (i, 0))],
        out_specs=pl.BlockSpec((1, 1), lambda i: (0, 0),
                               memory_space=pltpu.MemorySpace.SMEM),
        out_shape=jax.ShapeDtypeStruct((1, 1), _f32),
    )(logits)


def _t2_body(l_ref, t_ref, s_ref, renorm_ref, key_ref):
    xbar = s_ref[0, 0] * (1.0 / N)
    sparsity = jnp.float32(K / N)
    r = sparsity / xbar
    beta = (1.0 - sparsity) / (1.0 - xbar)
    prob = jax.nn.sigmoid(5.0 * l_ref[...])
    renorm = jnp.where(r <= 1.0, prob * r, 1.0 - (1.0 - prob) * beta)
    renorm_ref[...] = renorm
    sm = jax.nn.sigmoid(12.0 * (renorm - t_ref[...]))
    key_ref[...] = lax.bitcast_convert_type(sm, _i32)


def _t2(logits, thresh, xsum):
    return pl.pallas_call(
        _t2_body,
        grid=(GRID,),
        in_specs=[
            pl.BlockSpec((ROWS_BLK, W), lambda i: (i, 0)),
            pl.BlockSpec((ROWS_BLK, W), lambda i: (i, 0)),
            pl.BlockSpec((1, 1), lambda i: (0, 0),
                         memory_space=pltpu.MemorySpace.SMEM),
        ],
        out_specs=[
            pl.BlockSpec((ROWS_BLK, W), lambda i: (i, 0)),
            pl.BlockSpec((ROWS_BLK, W), lambda i: (i, 0)),
        ],
        out_shape=[
            jax.ShapeDtypeStruct((H, W), _f32),
            jax.ShapeDtypeStruct((H, W), _i32),
        ],
    )(logits, thresh, xsum)


def _sc_level(level, nb):
    mesh = plsc.VectorSubcoreMesh(core_axis_name="c", subcore_axis_name="s")

    def body(*args):
        if level == 1:
            keys_hbm, hist_hbm, kbuf0, kbuf1, hist, red, sem0, sem1 = args
            sbuf = None
        else:
            (keys_hbm, state_hbm, hist_hbm, kbuf0, kbuf1, sbuf, hist, red,
             sem0, sem1) = args
        wid = lax.axis_index("s") * NC + lax.axis_index("c")
        base = wid * PER_W
        lane = lax.iota(_i32, LANES)
        ones = jnp.ones((LANES,), _i32)

        if level > 1:
            pltpu.sync_copy(state_hbm, sbuf)
            sv = sbuf[pl.ds(0, LANES)]
            p1 = sv[0]
            p2 = sv[1]
            p3 = sv[2]
            p4 = sv[3]
            q2 = (p1 << 12) | p2
            v_full = (q2 << 6) | (p3 >> 6)
            j18 = ((p3 & 0x3F) << 12) | p4

        def zero_body(i, _):
            hist[pl.ds(i * LANES, LANES)] = jnp.zeros((LANES,), _i32)
            return 0

        lax.fori_loop(0, (LANES * nb) // LANES, zero_body, 0)

        lane_nb = lane * nb
        jconst = (JMAX - base) - lane

        UNROLL = 8

        def process(cbuf, c):
            def vec_body(vb, _):
                for u in range(UNROLL):
                    v = vb * UNROLL + u
                    kv = cbuf[v // VPR, pl.ds((v % VPR) * LANES, LANES)]
                    jdx = jconst - (c * CH + v * LANES)
                    if level == 1:
                        bucket = kv >> 18
                        pred = None
                    elif level == 2:
                        bucket = (kv >> 6) & 0xFFF
                        pred = (kv >> 18) == p1
                    elif level == 3:
                        bucket = ((kv & 0x3F) << 6) | (jdx >> 16)
                        pred = (kv >> 6) == q2
                    elif level == 4:
                        bucket = (jdx >> 4) & 0xFFF
                        pred = (kv == v_full) & ((jdx >> 16) == (p3 & 0x3F))
                    else:
                        bucket = jdx & 0xF
                        pred = (kv == v_full) & ((jdx >> 4) == j18)
                    plsc.addupdate_scatter(hist, [lane_nb + bucket], ones,
                                           mask=pred)
                return 0

            lax.fori_loop(0, CH // LANES // UNROLL, vec_body, 0)

        rbase = wid * ROWS_W
        pltpu.async_copy(keys_hbm.at[pl.ds(rbase, CHR)], kbuf0, sem0)

        def pair_body(p, _):
            c0 = p * 2
            pltpu.async_copy(
                keys_hbm.at[pl.ds(rbase + (c0 + 1) * CHR, CHR)], kbuf1, sem1)
            pltpu.make_async_copy(
                keys_hbm.at[pl.ds(rbase + c0 * CHR, CHR)], kbuf0, sem0).wait()
            process(kbuf0, c0)

            @pl.when(p + 1 < NPAIR)
            def _():
                pltpu.async_copy(
                    keys_hbm.at[pl.ds(rbase + (c0 + 2) * CHR, CHR)], kbuf0,
                    sem0)

            pltpu.make_async_copy(
                keys_hbm.at[pl.ds(rbase + (c0 + 1) * CHR, CHR)], kbuf1,
                sem1).wait()
            process(kbuf1, c0 + 1)
            return 0

        lax.fori_loop(0, NPAIR, pair_body, 0)

        def red_body(g, _):
            def lane_body(l, acc):
                return acc + hist[pl.ds(l * nb + g * LANES, LANES)]

            acc = lax.fori_loop(0, LANES, lane_body, jnp.zeros((LANES,), _i32))
            red[0, pl.ds(g * LANES, LANES)] = acc
            return 0

        lax.fori_loop(0, nb // LANES, red_body, 0)
        pltpu.sync_copy(red, hist_hbm.at[pl.ds(wid, 1)])

    scratch = [
        pltpu.VMEM((CHR, W), _i32),
        pltpu.VMEM((CHR, W), _i32),
        pltpu.VMEM((LANES * nb,), _i32),
        pltpu.VMEM((1, nb), _i32),
        pltpu.SemaphoreType.DMA,
        pltpu.SemaphoreType.DMA,
    ]
    if level > 1:
        scratch.insert(2, pltpu.VMEM((128,), _i32))
    return pl.kernel(
        body,
        out_type=jax.ShapeDtypeStruct((NW, nb), _i32),
        mesh=mesh,
        scratch_types=scratch,
        compiler_params=pltpu.CompilerParams(needs_layout_passes=False,
                                             disable_bounds_checks=True),
    )


def _merge_body(slot, bits, nb, is_l3, hist_ref, sin_ref, sout_ref):
    krem = sin_ref[5]
    acc = jnp.sum(hist_ref[...], axis=0, keepdims=True)
    bid = lax.broadcasted_iota(_i32, (1, nb), 1)
    cand = jnp.int32(0)
    for bit in reversed(range(bits)):
        t = cand | (1 << bit)
        c = jnp.sum(jnp.where(bid >= t, acc, 0))
        cand = jnp.where(c >= krem, t, cand)
    gt = jnp.sum(jnp.where(bid > cand, acc, 0))
    krem_new = krem - gt
    for j in range(8):
        sout_ref[j] = sin_ref[j]
    sout_ref[slot] = cand
    sout_ref[5] = krem_new
    if is_l3:
        e_cnt = jnp.sum(jnp.where(bid == cand, acc, 0))
        sout_ref[3] = 0
        sout_ref[4] = 0
        sout_ref[7] = (e_cnt > krem_new).astype(_i32)


def _merge(hist, state, slot, bits, nb, is_l3=False):
    return pl.pallas_call(
        functools.partial(_merge_body, slot, bits, nb, is_l3),
        in_specs=[
            pl.BlockSpec(memory_space=pltpu.MemorySpace.VMEM),
            pl.BlockSpec(memory_space=pltpu.MemorySpace.SMEM),
        ],
        out_specs=pl.BlockSpec(memory_space=pltpu.MemorySpace.SMEM),
        out_shape=jax.ShapeDtypeStruct((128,), _i32),
    )(hist, state)


def _t3_body(k_ref, s_ref, o_ref):
    p1 = s_ref[0]
    p2 = s_ref[1]
    p3 = s_ref[2]
    p4 = s_ref[3]
    p5 = s_ref[4]
    v_full = (((p1 << 12) | p2) << 6) | (p3 >> 6)
    j_thr = ((p3 & 0x3F) << 16) | (p4 << 4) | p5
    i0 = pl.program_id(0)
    r = lax.broadcasted_iota(_i32, (ROWS_BLK, W), 0) + i0 * ROWS_BLK
    c = lax.broadcasted_iota(_i32, (ROWS_BLK, W), 1)
    jdx = JMAX - (r * W + c)
    kv = k_ref[...]
    sel = (kv > v_full) | ((kv == v_full) & (jdx >= j_thr))
    o_ref[...] = sel.astype(_f32)


def _t3(keys2d, state):
    return pl.pallas_call(
        _t3_body,
        grid=(GRID,),
        in_specs=[
            pl.BlockSpec((ROWS_BLK, W), lambda i: (i, 0)),
            pl.BlockSpec((128,), lambda i: (0,),
                         memory_space=pltpu.MemorySpace.SMEM),
        ],
        out_specs=pl.BlockSpec((ROWS_BLK, W), lambda i: (i, 0)),
        out_shape=jax.ShapeDtypeStruct((H, W), _f32),
    )(keys2d, state)


@functools.lru_cache(maxsize=None)
def _sc(level, nb):
    return _sc_level(level, nb)


def kernel(x, prob_mask_logits, thresh):
    del x
    xsum = _t1(prob_mask_logits)
    renorm, keys2d = _t2(prob_mask_logits, thresh, xsum)

    state0 = jnp.zeros((128,), _i32).at[5].set(K)
    s1 = _merge(_sc(1, NB)(keys2d), state0, slot=0, bits=12, nb=NB)
    s2 = _merge(_sc(2, NB)(keys2d, s1), s1, slot=1, bits=12, nb=NB)
    s3 = _merge(_sc(3, NB)(keys2d, s2), s2, slot=2, bits=12, nb=NB,
                is_l3=True)

    def tie_path(args):
        kf, s = args
        s4 = _merge(_sc(4, NB)(kf, s), s, slot=3, bits=12, nb=NB)
        s5 = _merge(_sc(5, NB5)(kf, s4), s4, slot=4, bits=4, nb=NB5)
        return s5

    sfin = lax.cond(s3[7] > 0, tie_path, lambda a: a[1], (keys2d, s3))
    hard = _t3(keys2d, sfin)
    return hard, renorm


# speedup vs baseline: 2.0281x; 1.7687x over previous
import functools

import jax
import jax.numpy as jnp
from jax import lax
from jax.experimental import pallas as pl
from jax.experimental.pallas import tpu as pltpu
from jax.experimental.pallas import tpu_sc as plsc

H, W = 2048, 2048
N = H * W
K = N // 4
JMAX = N - 1

NC, NS, LANES = 2, 16, 16
NW = NC * NS
PER_W = N // NW
ROWS_W = PER_W // W
CH = 16384
CHR = CH // W
VPR = W // LANES
NCHUNK = PER_W // CH
NPAIR = NCHUNK // 2

NB = 4096
NB5 = 16

ROWS_BLK = 128
GRID = H // ROWS_BLK

_f32 = jnp.float32
_i32 = jnp.int32


def _t1_body(x_ref, s_ref):
    @pl.when(pl.program_id(0) == 0)
    def _():
        s_ref[0, 0] = 0.0

    s_ref[0, 0] += jnp.sum(jax.nn.sigmoid(5.0 * x_ref[...]))


def _t1(logits):
    return pl.pallas_call(
        _t1_body,
        grid=(GRID,),
        in_specs=[pl.BlockSpec((ROWS_BLK, W), lambda i: (i, 0))],
        out_specs=pl.BlockSpec((1, 1), lambda i: (0, 0),
                               memory_space=pltpu.MemorySpace.SMEM),
        out_shape=jax.ShapeDtypeStruct((1, 1), _f32),
    )(logits)


def _t2_body(l_ref, t_ref, s_ref, renorm_ref, key_ref):
    xbar = s_ref[0, 0] * (1.0 / N)
    sparsity = jnp.float32(K / N)
    r = sparsity / xbar
    beta = (1.0 - sparsity) / (1.0 - xbar)
    prob = jax.nn.sigmoid(5.0 * l_ref[...])
    renorm = jnp.where(r <= 1.0, prob * r, 1.0 - (1.0 - prob) * beta)
    renorm_ref[...] = renorm
    sm = jax.nn.sigmoid(12.0 * (renorm - t_ref[...]))
    key_ref[...] = lax.bitcast_convert_type(sm, _i32)


def _t2(logits, thresh, xsum):
    return pl.pallas_call(
        _t2_body,
        grid=(GRID,),
        in_specs=[
            pl.BlockSpec((ROWS_BLK, W), lambda i: (i, 0)),
            pl.BlockSpec((ROWS_BLK, W), lambda i: (i, 0)),
            pl.BlockSpec((1, 1), lambda i: (0, 0),
                         memory_space=pltpu.MemorySpace.SMEM),
        ],
        out_specs=[
            pl.BlockSpec((ROWS_BLK, W), lambda i: (i, 0)),
            pl.BlockSpec((ROWS_BLK, W), lambda i: (i, 0)),
        ],
        out_shape=[
            jax.ShapeDtypeStruct((H, W), _f32),
            jax.ShapeDtypeStruct((H, W), _i32),
        ],
    )(logits, thresh, xsum)


def _sc_level(level, nb):
    mesh = plsc.VectorSubcoreMesh(core_axis_name="c", subcore_axis_name="s")

    def body(*args):
        if level == 1:
            keys_hbm, hist_hbm, kbuf0, kbuf1, hist, red, sem0, sem1 = args
            sbuf = None
        else:
            (keys_hbm, state_hbm, hist_hbm, kbuf0, kbuf1, sbuf, hist, red,
             sem0, sem1) = args
        wid = lax.axis_index("s") * NC + lax.axis_index("c")
        base = wid * PER_W
        lane = lax.iota(_i32, LANES)
        ones = jnp.ones((LANES,), _i32)

        if level > 1:
            pltpu.sync_copy(state_hbm, sbuf)
            sv = sbuf[pl.ds(0, LANES)]
            p1 = sv[0]
            p2 = sv[1]
            p3 = sv[2]
            p4 = sv[3]
            q2 = (p1 << 12) | p2
            v_full = (q2 << 6) | (p3 >> 6)
            j18 = ((p3 & 0x3F) << 12) | p4

        def zero_body(i, _):
            hist[pl.ds(i * LANES, LANES)] = jnp.zeros((LANES,), _i32)
            return 0

        lax.fori_loop(0, (LANES * nb) // LANES, zero_body, 0)

        lane_nb = lane * nb
        jconst = (JMAX - base) - lane

        def process(cbuf, c):
            @plsc.parallel_loop(0, CH // LANES, unroll=8)
            def vec_body(v):
                kv = cbuf[v // VPR, pl.ds((v % VPR) * LANES, LANES)]
                jdx = jconst - (c * CH + v * LANES)
                if level == 1:
                    bucket = kv >> 18
                    pred = None
                elif level == 2:
                    bucket = (kv >> 6) & 0xFFF
                    pred = (kv >> 18) == p1
                elif level == 3:
                    bucket = ((kv & 0x3F) << 6) | (jdx >> 16)
                    pred = (kv >> 6) == q2
                elif level == 4:
                    bucket = (jdx >> 4) & 0xFFF
                    pred = (kv == v_full) & ((jdx >> 16) == (p3 & 0x3F))
                else:
                    bucket = jdx & 0xF
                    pred = (kv == v_full) & ((jdx >> 4) == j18)
                plsc.addupdate_scatter(hist, [lane_nb + bucket], ones,
                                       mask=pred)

        rbase = wid * ROWS_W
        pltpu.async_copy(keys_hbm.at[pl.ds(rbase, CHR)], kbuf0, sem0)

        def pair_body(p, _):
            c0 = p * 2
            pltpu.async_copy(
                keys_hbm.at[pl.ds(rbase + (c0 + 1) * CHR, CHR)], kbuf1, sem1)
            pltpu.make_async_copy(
                keys_hbm.at[pl.ds(rbase + c0 * CHR, CHR)], kbuf0, sem0).wait()
            process(kbuf0, c0)

            @pl.when(p + 1 < NPAIR)
            def _():
                pltpu.async_copy(
                    keys_hbm.at[pl.ds(rbase + (c0 + 2) * CHR, CHR)], kbuf0,
                    sem0)

            pltpu.make_async_copy(
                keys_hbm.at[pl.ds(rbase + (c0 + 1) * CHR, CHR)], kbuf1,
                sem1).wait()
            process(kbuf1, c0 + 1)
            return 0

        lax.fori_loop(0, NPAIR, pair_body, 0)

        def red_body(g, _):
            def lane_body(l, acc):
                return acc + hist[pl.ds(l * nb + g * LANES, LANES)]

            acc = lax.fori_loop(0, LANES, lane_body, jnp.zeros((LANES,), _i32))
            red[0, pl.ds(g * LANES, LANES)] = acc
            return 0

        lax.fori_loop(0, nb // LANES, red_body, 0)
        pltpu.sync_copy(red, hist_hbm.at[pl.ds(wid, 1)])

    scratch = [
        pltpu.VMEM((CHR, W), _i32),
        pltpu.VMEM((CHR, W), _i32),
        pltpu.VMEM((LANES * nb,), _i32),
        pltpu.VMEM((1, nb), _i32),
        pltpu.SemaphoreType.DMA,
        pltpu.SemaphoreType.DMA,
    ]
    if level > 1:
        scratch.insert(2, pltpu.VMEM((128,), _i32))
    return pl.kernel(
        body,
        out_type=jax.ShapeDtypeStruct((NW, nb), _i32),
        mesh=mesh,
        scratch_types=scratch,
        compiler_params=pltpu.CompilerParams(needs_layout_passes=False,
                                             disable_bounds_checks=True),
    )


def _merge_body(slot, bits, nb, is_l3, hist_ref, sin_ref, sout_ref):
    krem = sin_ref[5]
    acc = jnp.sum(hist_ref[...], axis=0, keepdims=True)
    bid = lax.broadcasted_iota(_i32, (1, nb), 1)
    cand = jnp.int32(0)
    for bit in reversed(range(bits)):
        t = cand | (1 << bit)
        c = jnp.sum(jnp.where(bid >= t, acc, 0))
        cand = jnp.where(c >= krem, t, cand)
    gt = jnp.sum(jnp.where(bid > cand, acc, 0))
    krem_new = krem - gt
    for j in range(8):
        sout_ref[j] = sin_ref[j]
    sout_ref[slot] = cand
    sout_ref[5] = krem_new
    if is_l3:
        e_cnt = jnp.sum(jnp.where(bid == cand, acc, 0))
        sout_ref[3] = 0
        sout_ref[4] = 0
        sout_ref[7] = (e_cnt > krem_new).astype(_i32)


def _merge(hist, state, slot, bits, nb, is_l3=False):
    return pl.pallas_call(
        functools.partial(_merge_body, slot, bits, nb, is_l3),
        in_specs=[
            pl.BlockSpec(memory_space=pltpu.MemorySpace.VMEM),
            pl.BlockSpec(memory_space=pltpu.MemorySpace.SMEM),
        ],
        out_specs=pl.BlockSpec(memory_space=pltpu.MemorySpace.SMEM),
        out_shape=jax.ShapeDtypeStruct((128,), _i32),
    )(hist, state)


def _t3_body(k_ref, s_ref, o_ref):
    p1 = s_ref[0]
    p2 = s_ref[1]
    p3 = s_ref[2]
    p4 = s_ref[3]
    p5 = s_ref[4]
    v_full = (((p1 << 12) | p2) << 6) | (p3 >> 6)
    j_thr = ((p3 & 0x3F) << 16) | (p4 << 4) | p5
    i0 = pl.program_id(0)
    r = lax.broadcasted_iota(_i32, (ROWS_BLK, W), 0) + i0 * ROWS_BLK
    c = lax.broadcasted_iota(_i32, (ROWS_BLK, W), 1)
    jdx = JMAX - (r * W + c)
    kv = k_ref[...]
    sel = (kv > v_full) | ((kv == v_full) & (jdx >= j_thr))
    o_ref[...] = sel.astype(_f32)


def _t3(keys2d, state):
    return pl.pallas_call(
        _t3_body,
        grid=(GRID,),
        in_specs=[
            pl.BlockSpec((ROWS_BLK, W), lambda i: (i, 0)),
            pl.BlockSpec((128,), lambda i: (0,),
                         memory_space=pltpu.MemorySpace.SMEM),
        ],
        out_specs=pl.BlockSpec((ROWS_BLK, W), lambda i: (i, 0)),
        out_shape=jax.ShapeDtypeStruct((H, W), _f32),
    )(keys2d, state)


@functools.lru_cache(maxsize=None)
def _sc(level, nb):
    return _sc_level(level, nb)


def kernel(x, prob_mask_logits, thresh):
    del x
    xsum = _t1(prob_mask_logits)
    renorm, keys2d = _t2(prob_mask_logits, thresh, xsum)

    state0 = jnp.zeros((128,), _i32).at[5].set(K)
    s1 = _merge(_sc(1, NB)(keys2d), state0, slot=0, bits=12, nb=NB)
    s2 = _merge(_sc(2, NB)(keys2d, s1), s1, slot=1, bits=12, nb=NB)
    s3 = _merge(_sc(3, NB)(keys2d, s2), s2, slot=2, bits=12, nb=NB,
                is_l3=True)

    def tie_path(args):
        kf, s = args
        s4 = _merge(_sc(4, NB)(kf, s), s, slot=3, bits=12, nb=NB)
        s5 = _merge(_sc(5, NB5)(kf, s4), s4, slot=4, bits=4, nb=NB5)
        return s5

    sfin = lax.cond(s3[7] > 0, tie_path, lambda a: a[1], (keys2d, s3))
    hard = _t3(keys2d, sfin)
    return hard, renorm


# speedup vs baseline: 2.7016x; 1.3320x over previous
import functools

import jax
import jax.numpy as jnp
from jax import lax
from jax.experimental import pallas as pl
from jax.experimental.pallas import tpu as pltpu
from jax.experimental.pallas import tpu_sc as plsc

H, W = 2048, 2048
N = H * W
K = N // 4
JMAX = N - 1

NC, NS, LANES = 2, 16, 16
NW = NC * NS
PER_W = N // NW
ROWS_W = PER_W // W
CH = 16384
CHR = CH // W
VPR = W // LANES
NCHUNK = PER_W // CH
NPAIR = NCHUNK // 2

NB = 4096
NB5 = 16

ROWS_BLK = 128
GRID = H // ROWS_BLK

_f32 = jnp.float32
_i32 = jnp.int32


def _t1_body(x_ref, s_ref):
    @pl.when(pl.program_id(0) == 0)
    def _():
        s_ref[0, 0] = 0.0

    s_ref[0, 0] += jnp.sum(jax.nn.sigmoid(5.0 * x_ref[...]))


def _t1(logits):
    return pl.pallas_call(
        _t1_body,
        grid=(GRID,),
        in_specs=[pl.BlockSpec((ROWS_BLK, W), lambda i: (i, 0))],
        out_specs=pl.BlockSpec((1, 1), lambda i: (0, 0),
                               memory_space=pltpu.MemorySpace.SMEM),
        out_shape=jax.ShapeDtypeStruct((1, 1), _f32),
    )(logits)


def _t2_body(l_ref, t_ref, s_ref, renorm_ref, key_ref):
    xbar = s_ref[0, 0] * (1.0 / N)
    sparsity = jnp.float32(K / N)
    r = sparsity / xbar
    beta = (1.0 - sparsity) / (1.0 - xbar)
    prob = jax.nn.sigmoid(5.0 * l_ref[...])
    renorm = jnp.where(r <= 1.0, prob * r, 1.0 - (1.0 - prob) * beta)
    renorm_ref[...] = renorm
    sm = jax.nn.sigmoid(12.0 * (renorm - t_ref[...]))
    key_ref[...] = lax.bitcast_convert_type(sm, _i32)


def _t2(logits, thresh, xsum):
    return pl.pallas_call(
        _t2_body,
        grid=(GRID,),
        in_specs=[
            pl.BlockSpec((ROWS_BLK, W), lambda i: (i, 0)),
            pl.BlockSpec((ROWS_BLK, W), lambda i: (i, 0)),
            pl.BlockSpec((1, 1), lambda i: (0, 0),
                         memory_space=pltpu.MemorySpace.SMEM),
        ],
        out_specs=[
            pl.BlockSpec((ROWS_BLK, W), lambda i: (i, 0)),
            pl.BlockSpec((ROWS_BLK, W), lambda i: (i, 0)),
        ],
        out_shape=[
            jax.ShapeDtypeStruct((H, W), _f32),
            jax.ShapeDtypeStruct((H, W), _i32),
        ],
    )(logits, thresh, xsum)


def _sc_level(level, nb):
    mesh = plsc.VectorSubcoreMesh(core_axis_name="c", subcore_axis_name="s")

    def body(*args):
        if level == 1:
            keys_hbm, hist_hbm, kbuf0, kbuf1, hist, red, sem0, sem1 = args
            sbuf = None
        else:
            (keys_hbm, state_hbm, hist_hbm, kbuf0, kbuf1, sbuf, hist, red,
             sem0, sem1) = args
        wid = lax.axis_index("s") * NC + lax.axis_index("c")
        base = wid * PER_W
        lane = lax.iota(_i32, LANES)
        ones = jnp.ones((LANES,), _i32)

        if level > 1:
            pltpu.sync_copy(state_hbm, sbuf)
            sv = sbuf[pl.ds(0, LANES)]
            p1 = sv[0]
            p2 = sv[1]
            p3 = sv[2]
            p4 = sv[3]
            q2 = (p1 << 12) | p2
            v_full = (q2 << 6) | (p3 >> 6)
            j18 = ((p3 & 0x3F) << 12) | p4

        @plsc.parallel_loop(0, (LANES * nb) // LANES, unroll=8)
        def zero_body(i):
            hist[pl.ds(i * LANES, LANES)] = jnp.zeros((LANES,), _i32)

        lane_nb = lane * nb
        jconst = (JMAX - base) - lane

        def process(cbuf, c):
            @plsc.parallel_loop(0, CH // LANES, unroll=8)
            def vec_body(v):
                kv = cbuf[v // VPR, pl.ds((v % VPR) * LANES, LANES)]
                jdx = jconst - (c * CH + v * LANES)
                if level == 1:
                    bucket = kv >> 18
                    pred = None
                elif level == 2:
                    bucket = (kv >> 6) & 0xFFF
                    pred = (kv >> 18) == p1
                elif level == 3:
                    bucket = ((kv & 0x3F) << 6) | (jdx >> 16)
                    pred = (kv >> 6) == q2
                elif level == 4:
                    bucket = (jdx >> 4) & 0xFFF
                    pred = (kv == v_full) & ((jdx >> 16) == (p3 & 0x3F))
                else:
                    bucket = jdx & 0xF
                    pred = (kv == v_full) & ((jdx >> 4) == j18)
                plsc.addupdate_scatter(hist, [lane_nb + bucket], ones,
                                       mask=pred)

        rbase = wid * ROWS_W
        pltpu.async_copy(keys_hbm.at[pl.ds(rbase, CHR)], kbuf0, sem0)

        def pair_body(p, _):
            c0 = p * 2
            pltpu.async_copy(
                keys_hbm.at[pl.ds(rbase + (c0 + 1) * CHR, CHR)], kbuf1, sem1)
            pltpu.make_async_copy(
                keys_hbm.at[pl.ds(rbase + c0 * CHR, CHR)], kbuf0, sem0).wait()
            process(kbuf0, c0)

            @pl.when(p + 1 < NPAIR)
            def _():
                pltpu.async_copy(
                    keys_hbm.at[pl.ds(rbase + (c0 + 2) * CHR, CHR)], kbuf0,
                    sem0)

            pltpu.make_async_copy(
                keys_hbm.at[pl.ds(rbase + (c0 + 1) * CHR, CHR)], kbuf1,
                sem1).wait()
            process(kbuf1, c0 + 1)
            return 0

        lax.fori_loop(0, NPAIR, pair_body, 0)

        @plsc.parallel_loop(0, nb // LANES, unroll=2)
        def red_body(g):
            acc = hist[pl.ds(g * LANES, LANES)]
            for l in range(1, LANES):
                acc = acc + hist[pl.ds(l * nb + g * LANES, LANES)]
            red[0, pl.ds(g * LANES, LANES)] = acc
        pltpu.sync_copy(red, hist_hbm.at[pl.ds(wid, 1)])

    scratch = [
        pltpu.VMEM((CHR, W), _i32),
        pltpu.VMEM((CHR, W), _i32),
        pltpu.VMEM((LANES * nb,), _i32),
        pltpu.VMEM((1, nb), _i32),
        pltpu.SemaphoreType.DMA,
        pltpu.SemaphoreType.DMA,
    ]
    if level > 1:
        scratch.insert(2, pltpu.VMEM((128,), _i32))
    return pl.kernel(
        body,
        out_type=jax.ShapeDtypeStruct((NW, nb), _i32),
        mesh=mesh,
        scratch_types=scratch,
        compiler_params=pltpu.CompilerParams(needs_layout_passes=False,
                                             disable_bounds_checks=True),
    )


def _merge_body(slot, bits, nb, is_l3, hist_ref, sin_ref, sout_ref):
    krem = sin_ref[5]
    acc = jnp.sum(hist_ref[...], axis=0, keepdims=True)
    bid = lax.broadcasted_iota(_i32, (1, nb), 1)
    cand = jnp.int32(0)
    for bit in reversed(range(bits)):
        t = cand | (1 << bit)
        c = jnp.sum(jnp.where(bid >= t, acc, 0))
        cand = jnp.where(c >= krem, t, cand)
    gt = jnp.sum(jnp.where(bid > cand, acc, 0))
    krem_new = krem - gt
    for j in range(8):
        sout_ref[j] = sin_ref[j]
    sout_ref[slot] = cand
    sout_ref[5] = krem_new
    if is_l3:
        e_cnt = jnp.sum(jnp.where(bid == cand, acc, 0))
        sout_ref[3] = 0
        sout_ref[4] = 0
        sout_ref[7] = (e_cnt > krem_new).astype(_i32)


def _merge(hist, state, slot, bits, nb, is_l3=False):
    return pl.pallas_call(
        functools.partial(_merge_body, slot, bits, nb, is_l3),
        in_specs=[
            pl.BlockSpec(memory_space=pltpu.MemorySpace.VMEM),
            pl.BlockSpec(memory_space=pltpu.MemorySpace.SMEM),
        ],
        out_specs=pl.BlockSpec(memory_space=pltpu.MemorySpace.SMEM),
        out_shape=jax.ShapeDtypeStruct((128,), _i32),
    )(hist, state)


def _t3_body(k_ref, s_ref, o_ref):
    p1 = s_ref[0]
    p2 = s_ref[1]
    p3 = s_ref[2]
    p4 = s_ref[3]
    p5 = s_ref[4]
    v_full = (((p1 << 12) | p2) << 6) | (p3 >> 6)
    j_thr = ((p3 & 0x3F) << 16) | (p4 << 4) | p5
    i0 = pl.program_id(0)
    r = lax.broadcasted_iota(_i32, (ROWS_BLK, W), 0) + i0 * ROWS_BLK
    c = lax.broadcasted_iota(_i32, (ROWS_BLK, W), 1)
    jdx = JMAX - (r * W + c)
    kv = k_ref[...]
    sel = (kv > v_full) | ((kv == v_full) & (jdx >= j_thr))
    o_ref[...] = sel.astype(_f32)


def _t3(keys2d, state):
    return pl.pallas_call(
        _t3_body,
        grid=(GRID,),
        in_specs=[
            pl.BlockSpec((ROWS_BLK, W), lambda i: (i, 0)),
            pl.BlockSpec((128,), lambda i: (0,),
                         memory_space=pltpu.MemorySpace.SMEM),
        ],
        out_specs=pl.BlockSpec((ROWS_BLK, W), lambda i: (i, 0)),
        out_shape=jax.ShapeDtypeStruct((H, W), _f32),
    )(keys2d, state)


@functools.lru_cache(maxsize=None)
def _sc(level, nb):
    return _sc_level(level, nb)


def kernel(x, prob_mask_logits, thresh):
    del x
    xsum = _t1(prob_mask_logits)
    renorm, keys2d = _t2(prob_mask_logits, thresh, xsum)

    state0 = jnp.zeros((128,), _i32).at[5].set(K)
    s1 = _merge(_sc(1, NB)(keys2d), state0, slot=0, bits=12, nb=NB)
    s2 = _merge(_sc(2, NB)(keys2d, s1), s1, slot=1, bits=12, nb=NB)
    s3 = _merge(_sc(3, NB)(keys2d, s2), s2, slot=2, bits=12, nb=NB,
                is_l3=True)

    def tie_path(args):
        kf, s = args
        s4 = _merge(_sc(4, NB)(kf, s), s, slot=3, bits=12, nb=NB)
        s5 = _merge(_sc(5, NB5)(kf, s4), s4, slot=4, bits=4, nb=NB5)
        return s5

    sfin = lax.cond(s3[7] > 0, tie_path, lambda a: a[1], (keys2d, s3))
    hard = _t3(keys2d, sfin)
    return hard, renorm
